# baseline (speedup 1.0000x reference)
import functools

import jax
import jax.numpy as jnp
from jax.experimental import pallas as pl
from jax.experimental.pallas import tpu as pltpu

_NBUF = 3


def _body(x_ref, adj_hbm, w1_ref, b1_ref, w2_ref, b2_ref, out_hbm,
          abuf, s1_ref, s2_ref, obuf, sems, osems, *, bm, g):
    def dma(slot, blk):
        return pltpu.make_async_copy(
            adj_hbm.at[pl.ds(blk * bm, bm), :], abuf.at[slot], sems.at[slot])

    def odma(slot, blk):
        return pltpu.make_async_copy(
            obuf.at[slot], out_hbm.at[pl.ds(blk * bm, bm), :], osems.at[slot])

    for slot in range(min(_NBUF, 2 * g)):
        dma(slot, slot % g).start()

    s1_ref[...] = jnp.dot(x_ref[...], w1_ref[...],
                          preferred_element_type=jnp.float32)

    for i in range(2 * g):
        slot = i % _NBUF
        dma(slot, i % g).wait()
        if i < g:
            h = jnp.dot(abuf[slot], s1_ref[...],
                        preferred_element_type=jnp.float32) + b1_ref[...]
            h = jnp.maximum(h, 0.0)
            s2_ref[i * bm:(i + 1) * bm, :] = jnp.dot(
                h, w2_ref[...], preferred_element_type=jnp.float32)
        else:
            k = i - g
            oslot = k % 2
            if k >= 2:
                odma(oslot, k - 2).wait()
            o2 = jnp.dot(abuf[slot], s2_ref[...],
                         preferred_element_type=jnp.float32) + b2_ref[...]
            shifted = o2 - jnp.max(o2, axis=-1, keepdims=True)
            lse = jnp.log(jnp.sum(jnp.exp(shifted), axis=-1, keepdims=True))
            obuf[oslot] = shifted - lse
            odma(oslot, k).start()
        nxt = i + _NBUF
        if nxt < 2 * g:
            dma(slot, nxt % g).start()

    for t in range(max(0, g - 2), g):
        odma(t % 2, t).wait()


def kernel(x, adj, W1, b1, W2, b2):
    n, nfeat = x.shape
    nhid = W1.shape[1]
    nclass = W2.shape[1]

    bm = next(b for b in (400, 200, 80, 40, 8) if n % b == 0)
    g = n // bm

    b1_2d = b1.reshape(1, nhid)
    b2_2d = b2.reshape(1, nclass)

    vmem = lambda: pl.BlockSpec(memory_space=pltpu.VMEM)
    out = pl.pallas_call(
        functools.partial(_body, bm=bm, g=g),
        in_specs=[
            vmem(),
            pl.BlockSpec(memory_space=pl.ANY),
            vmem(), vmem(), vmem(), vmem(),
        ],
        out_specs=pl.BlockSpec(memory_space=pl.ANY),
        out_shape=jax.ShapeDtypeStruct((n, nclass), jnp.float32),
        scratch_shapes=[
            pltpu.VMEM((_NBUF, bm, n), jnp.float32),
            pltpu.VMEM((n, nhid), jnp.float32),
            pltpu.VMEM((n, nclass), jnp.float32),
            pltpu.VMEM((2, bm, nclass), jnp.float32),
            pltpu.SemaphoreType.DMA((_NBUF,)),
            pltpu.SemaphoreType.DMA((2,)),
        ],
        compiler_params=pltpu.CompilerParams(
            vmem_limit_bytes=100 * 1024 * 1024,
        ),
    )(x, adj, W1, b1_2d, W2, b2_2d)
    return out


# speedup vs baseline: 1.0288x; 1.0288x over previous
import functools

import jax
import jax.numpy as jnp
from jax.experimental import pallas as pl
from jax.experimental.pallas import tpu as pltpu


def _body(x_ref, adj_ref, w1_ref, b1_ref, w2_ref, b2_ref, out_ref,
          s1_ref, s2_ref, *, bm, phase_steps):
    i = pl.program_id(0)

    @pl.when(i == 0)
    def _():
        s1_ref[...] = jnp.dot(x_ref[...], w1_ref[...],
                              preferred_element_type=jnp.float32)

    @pl.when(i < phase_steps)
    def _():
        h = jnp.dot(adj_ref[...], s1_ref[...],
                    preferred_element_type=jnp.float32) + b1_ref[...]
        h = jnp.maximum(h, 0.0)
        row = jnp.dot(h, w2_ref[...], preferred_element_type=jnp.float32)
        s2_ref[pl.ds(i * bm, bm), :] = row

    @pl.when(i >= phase_steps)
    def _():
        o = jnp.dot(adj_ref[...], s2_ref[...],
                    preferred_element_type=jnp.float32) + b2_ref[...]
        shifted = o - jnp.max(o, axis=-1, keepdims=True)
        lse = jnp.log(jnp.sum(jnp.exp(shifted), axis=-1, keepdims=True))
        out_ref[...] = shifted - lse


def kernel(x, adj, W1, b1, W2, b2):
    n, nfeat = x.shape
    nhid = W1.shape[1]
    nclass = W2.shape[1]

    bm = next(b for b in (400, 200, 80, 40, 8) if n % b == 0)
    phase_steps = n // bm
    grid = (2 * phase_steps,)

    b1_2d = b1.reshape(1, nhid)
    b2_2d = b2.reshape(1, nclass)

    out = pl.pallas_call(
        functools.partial(_body, bm=bm, phase_steps=phase_steps),
        grid=grid,
        in_specs=[
            pl.BlockSpec((n, nfeat), lambda i: (0, 0)),
            pl.BlockSpec((bm, n), lambda i, ps=phase_steps: (jax.lax.rem(i, ps), 0)),
            pl.BlockSpec((nfeat, nhid), lambda i: (0, 0)),
            pl.BlockSpec((1, nhid), lambda i: (0, 0)),
            pl.BlockSpec((nhid, nclass), lambda i: (0, 0)),
            pl.BlockSpec((1, nclass), lambda i: (0, 0)),
        ],
        out_specs=pl.BlockSpec(
            (bm, nclass),
            lambda i, ps=phase_steps: (jax.lax.max(i - ps, 0), 0)),
        out_shape=jax.ShapeDtypeStruct((n, nclass), jnp.float32),
        scratch_shapes=[
            pltpu.VMEM((n, nhid), jnp.float32),
            pltpu.VMEM((n, nclass), jnp.float32),
        ],
        compiler_params=pltpu.CompilerParams(
            dimension_semantics=("arbitrary",),
        ),
    )(x, adj, W1, b1_2d, W2, b2_2d)
    return out
